# baseline (speedup 1.0000x reference)
import functools

import jax
import jax.numpy as jnp
import numpy as np
from jax import lax
from jax.experimental import pallas as pl
from jax.experimental.pallas import tpu as pltpu
from jax.experimental.pallas import tpu_sc as plsc

F32 = jnp.float32
I32 = jnp.int32

NC = 2
NS = 16
NW = NC * NS
K = 128
G = 64


def _deg_body(src_hbm, dst_hbm, out_hbm, src_out, dst_out, idx_v, hist,
              *, nchunks):
    c = lax.axis_index("c")
    s = lax.axis_index("s")
    w = c * jnp.int32(NS) + s
    npad = hist.shape[0]
    pltpu.sync_copy(src_hbm.at[w], idx_v)
    pltpu.sync_copy(idx_v, src_out.at[w])
    pltpu.sync_copy(dst_hbm.at[w], idx_v)
    pltpu.sync_copy(idx_v, dst_out.at[w])

    def zero(r, carry):
        hist[pl.ds(r * jnp.int32(16), 16)] = jnp.zeros((16,), F32)
        return carry

    lax.fori_loop(jnp.int32(0), jnp.int32(npad // 16), zero, jnp.int32(0))

    def body(i, carry):
        for k in range(K // 16):
            v = idx_v[i, pl.ds(jnp.int32(k * 16), 16)]
            plsc.addupdate_scatter(hist, [v], jnp.ones((16,), F32))
        return carry

    lax.fori_loop(jnp.int32(0), jnp.int32(nchunks), body, jnp.int32(0))
    pltpu.sync_copy(hist, out_hbm.at[c, s])


def _agg_body(src_hbm, dst_hbm, y_hbm, out_hbm,
              sidx_h, didx_h, rows0, rows1, acc, sem0, sem1, *, nchunks):
    c = lax.axis_index("c")
    s = lax.axis_index("s")
    w = c * jnp.int32(NS) + s
    rows_per_sub = acc.shape[0] // NS

    def zrow(r, carry):
        for j in range(rows0.shape[1] // 16):
            rows0[r, pl.ds(jnp.int32(j * 16), 16)] = jnp.zeros((16,), F32)
        return carry

    lax.fori_loop(jnp.int32(0), jnp.int32(K), zrow, jnp.int32(0))
    for b in range(rows_per_sub // K):
        pltpu.sync_copy(
            rows0, acc.at[pl.ds(s * jnp.int32(rows_per_sub) + jnp.int32(b * K), K)])
    plsc.subcore_barrier()
    nh = nchunks // 2
    n2 = nh // 2

    def gat(i, rows, sem):
        pltpu.async_copy(y_hbm.at[sidx_h.at[i]], rows, sem)

    def gwait(i, rows, sem):
        pltpu.make_async_copy(y_hbm.at[sidx_h.at[i]], rows, sem).wait()

    def body(i2, carry):
        i0 = i2 * jnp.int32(2)
        i1 = i0 + jnp.int32(1)
        gat(i1, rows1, sem1)
        gwait(i0, rows0, sem0)
        pltpu.sync_copy(rows0, acc.at[didx_h.at[i0]], add=True)

        @pl.when(i2 < jnp.int32(n2 - 1))
        def _pref():
            gat(i0 + jnp.int32(2), rows0, sem0)

        gwait(i1, rows1, sem1)
        pltpu.sync_copy(rows1, acc.at[didx_h.at[i1]], add=True)
        return carry

    for half in range(2):
        pltpu.sync_copy(src_hbm.at[w, pl.ds(jnp.int32(half * nh), nh)], sidx_h)
        pltpu.sync_copy(dst_hbm.at[w, pl.ds(jnp.int32(half * nh), nh)], didx_h)
        gat(jnp.int32(0), rows0, sem0)
        lax.fori_loop(jnp.int32(0), jnp.int32(n2), body, jnp.int32(0))
    plsc.subcore_barrier()
    pltpu.sync_copy(acc.at[pl.ds(s * jnp.int32(rows_per_sub), rows_per_sub)],
                    out_hbm.at[c, pl.ds(s * jnp.int32(rows_per_sub), rows_per_sub)])


def _dinv_from_parts(degp_ref):
    deg = jnp.sum(degp_ref[...], axis=0) + 1.0
    return lax.rsqrt(deg)


def _mm_scale_kernel(x_ref, w_ref, degp_ref, o_ref):
    dinv = _dinv_from_parts(degp_ref)
    o_ref[...] = jnp.dot(x_ref[...], w_ref[...], preferred_element_type=F32, precision=lax.Precision.HIGHEST) * dinv


def _combine_kernel(aggp_ref, y_ref, degp_ref, b_ref, w_ref, o_ref):
    dinv = _dinv_from_parts(degp_ref)
    t = (aggp_ref[0] + aggp_ref[1] + y_ref[...]) * dinv + b_ref[...]
    h = jnp.maximum(t, 0.0)
    o_ref[...] = jnp.dot(h, w_ref[...], preferred_element_type=F32, precision=lax.Precision.HIGHEST) * dinv


def _final_kernel(aggp_ref, y_ref, degp_ref, b_ref, batch_ref, wout_ref,
                  bout_ref, o_ref, sums, cnts):
    i = pl.program_id(0)

    @pl.when(i == 0)
    def _init():
        sums[...] = jnp.zeros_like(sums)
        cnts[...] = jnp.zeros_like(cnts)

    dinv = _dinv_from_parts(degp_ref)
    t = (aggp_ref[0] + aggp_ref[1] + y_ref[...]) * dinv + b_ref[...]
    h = jnp.maximum(t, 0.0)
    oh = (batch_ref[...] == lax.broadcasted_iota(I32, (1, G), 1)).astype(F32)
    dn = (((0,), (0,)), ((), ()))
    sums[...] += lax.dot_general(oh, h, dn, preferred_element_type=F32, precision=lax.Precision.HIGHEST)
    cnts[...] += lax.dot_general(oh, jnp.ones_like(h), dn, preferred_element_type=F32, precision=lax.Precision.HIGHEST)

    @pl.when(i == pl.num_programs(0) - 1)
    def _fin():
        mean = sums[...] / jnp.maximum(cnts[...], 1.0)
        o_ref[...] = jnp.dot(mean, wout_ref[...], preferred_element_type=F32, precision=lax.Precision.HIGHEST) + bout_ref[...]


def _sc_mesh():
    return plsc.VectorSubcoreMesh(core_axis_name="c", subcore_axis_name="s",
                                  num_cores=NC, num_subcores=NS)


def _deg_parts(src_p, dst_p, npad, nch):
    call = pl.kernel(
        functools.partial(_deg_body, nchunks=nch),
        out_type=(jax.ShapeDtypeStruct((NC, NS, npad), F32),
                  jax.ShapeDtypeStruct((NW, nch, K), I32),
                  jax.ShapeDtypeStruct((NW, nch, K), I32)),
        mesh=_sc_mesh(),
        scratch_types=[
            pltpu.VMEM((nch, K), I32),
            pltpu.VMEM((npad,), F32),
        ],
        compiler_params=pltpu.CompilerParams(needs_layout_passes=False),
    )
    return call(src_p.reshape(NW, nch, K), dst_p.reshape(NW, nch, K))


def _agg_parts(src_p, dst_p, y, npad, nch):
    h = y.shape[1]
    call = pl.kernel(
        functools.partial(_agg_body, nchunks=nch),
        out_type=jax.ShapeDtypeStruct((NC, npad, h), F32),
        mesh=_sc_mesh(),
        scratch_types=[
            pltpu.VMEM((nch // 2, K), I32),
            pltpu.VMEM((nch // 2, K), I32),
            pltpu.VMEM((K, h), F32),
            pltpu.VMEM((K, h), F32),
            pltpu.VMEM_SHARED((npad, h), F32),
            pltpu.SemaphoreType.DMA,
            pltpu.SemaphoreType.DMA,
        ],
        compiler_params=pltpu.CompilerParams(needs_layout_passes=False),
    )
    return call(src_p, dst_p, y)


_Z = np.int32(0)


def _im_i0(i):
    return i, _Z


def _im_0i0(i):
    return _Z, i, _Z


def _im_00(i):
    return _Z, _Z


def kernel(x, edge_index, batch, W1, b1, W2, b2, Wout, bout):
    N, D = x.shape
    H = W1.shape[1]
    E = edge_index.shape[1]
    NPAD = (N // 1024 + 1) * 1024
    R = 1024
    grid = NPAD // R
    epc = NW * K
    nch = (-(-E // epc) + 3) // 4 * 4
    e_pad = nch * epc

    x32 = x.astype(F32)
    src_p = jnp.concatenate([edge_index[0].astype(I32),
                             jnp.zeros((e_pad - E,), I32)])
    pad_dst = N + jnp.arange(e_pad - E, dtype=I32) % jnp.int32(NPAD - N)
    dst_p = jnp.concatenate([edge_index[1].astype(I32), pad_dst])
    x_p = jnp.concatenate([x32, jnp.zeros((NPAD - N, D), F32)])
    batch_p = jnp.concatenate([batch.astype(I32),
                               jnp.full((NPAD - N,), G, I32)]).reshape(NPAD, 1)
    W1f = W1.astype(F32)
    W2f = W2.astype(F32)
    Woutf = Wout.astype(F32)
    b1r = b1.astype(F32).reshape(1, H)
    b2r = b2.astype(F32).reshape(1, H)
    boutr = bout.astype(F32).reshape(1, 1)

    degparts, src_l, dst_l = _deg_parts(src_p, dst_p, NPAD, nch)
    degparts = degparts.reshape(NW, NPAD, 1)

    y1 = pl.pallas_call(
        _mm_scale_kernel,
        grid=(grid,),
        in_specs=[
            pl.BlockSpec((R, D), _im_i0),
            pl.BlockSpec((D, H), _im_00),
            pl.BlockSpec((NW, R, 1), _im_0i0),
        ],
        out_specs=pl.BlockSpec((R, H), _im_i0),
        out_shape=jax.ShapeDtypeStruct((NPAD, H), F32),
    )(x_p, W1f, degparts)

    agg1 = _agg_parts(src_l, dst_l, y1, NPAD, nch)

    y2 = pl.pallas_call(
        _combine_kernel,
        grid=(grid,),
        in_specs=[
            pl.BlockSpec((NC, R, H), _im_0i0),
            pl.BlockSpec((R, H), _im_i0),
            pl.BlockSpec((NW, R, 1), _im_0i0),
            pl.BlockSpec((1, H), _im_00),
            pl.BlockSpec((H, H), _im_00),
        ],
        out_specs=pl.BlockSpec((R, H), _im_i0),
        out_shape=jax.ShapeDtypeStruct((NPAD, H), F32),
    )(agg1, y1, degparts, b1r, W2f)

    agg2 = _agg_parts(src_l, dst_l, y2, NPAD, nch)

    out = pl.pallas_call(
        _final_kernel,
        grid=(grid,),
        in_specs=[
            pl.BlockSpec((NC, R, H), _im_0i0),
            pl.BlockSpec((R, H), _im_i0),
            pl.BlockSpec((NW, R, 1), _im_0i0),
            pl.BlockSpec((1, H), _im_00),
            pl.BlockSpec((R, 1), _im_i0),
            pl.BlockSpec((H, 1), _im_00),
            pl.BlockSpec((1, 1), _im_00),
        ],
        out_specs=pl.BlockSpec((G, 1), _im_00),
        out_shape=jax.ShapeDtypeStruct((G, 1), F32),
        scratch_shapes=[
            pltpu.VMEM((G, H), F32),
            pltpu.VMEM((G, H), F32),
        ],
    )(agg2, y2, degparts, b2r, batch_p, Woutf, boutr)

    return out.astype(jnp.float64)


# speedup vs baseline: 2.4977x; 2.4977x over previous
import functools

import jax
import jax.numpy as jnp
import numpy as np
from jax import lax
from jax.experimental import pallas as pl
from jax.experimental.pallas import tpu as pltpu
from jax.experimental.pallas import tpu_sc as plsc

F32 = jnp.float32
I32 = jnp.int32

NC = 2
NS = 16
NW = NC * NS
K = 128
G = 64


def _deg_body(dst_hbm, out_hbm, idx_v, hist, *, nchunks, cw):
    c = lax.axis_index("c")
    s = lax.axis_index("s")
    w = c * jnp.int32(NS) + s
    npad = hist.shape[0]
    pltpu.sync_copy(dst_hbm.at[w], idx_v)

    def zero(r, carry):
        hist[pl.ds(r * jnp.int32(16), 16)] = jnp.zeros((16,), F32)
        return carry

    lax.fori_loop(jnp.int32(0), jnp.int32(npad // 16), zero, jnp.int32(0))

    def body(i, carry):
        for k in range(cw // 16):
            v = idx_v[i, pl.ds(jnp.int32(k * 16), 16)]
            plsc.addupdate_scatter(hist, [v], jnp.ones((16,), F32))
        return carry

    lax.fori_loop(jnp.int32(0), jnp.int32(nchunks), body, jnp.int32(0))
    pltpu.sync_copy(hist, out_hbm.at[c, s])


def _agg_body(src_hbm, dst_hbm, y_hbm, out_hbm,
              sidx_h, didx_h, rows0, rows1, acc, sem0, sem1, *, nchunks):
    c = lax.axis_index("c")
    s = lax.axis_index("s")
    w = c * jnp.int32(NS) + s
    rows_per_sub = acc.shape[0] // NS
    ka = rows0.shape[0]

    def zrow(r, carry):
        for j in range(rows0.shape[1] // 16):
            rows0[r, pl.ds(jnp.int32(j * 16), 16)] = jnp.zeros((16,), F32)
        return carry

    lax.fori_loop(jnp.int32(0), jnp.int32(ka), zrow, jnp.int32(0))
    nfull = rows_per_sub // ka
    for b in range(nfull):
        pltpu.sync_copy(
            rows0, acc.at[pl.ds(s * jnp.int32(rows_per_sub) + jnp.int32(b * ka), ka)])
    tail = rows_per_sub - nfull * ka
    if tail:
        pltpu.sync_copy(
            rows0.at[pl.ds(0, tail)],
            acc.at[pl.ds(s * jnp.int32(rows_per_sub) + jnp.int32(nfull * ka), tail)])
    plsc.subcore_barrier()
    nh = nchunks // 2
    n2 = nh // 2

    def gat(i, rows, sem):
        pltpu.async_copy(y_hbm.at[sidx_h.at[i]], rows, sem)

    def gwait(i, rows, sem):
        pltpu.make_async_copy(y_hbm.at[sidx_h.at[i]], rows, sem).wait()

    def body(i2, carry):
        i0 = i2 * jnp.int32(2)
        i1 = i0 + jnp.int32(1)
        gat(i1, rows1, sem1)
        gwait(i0, rows0, sem0)
        pltpu.sync_copy(rows0, acc.at[didx_h.at[i0]], add=True)

        @pl.when(i2 < jnp.int32(n2 - 1))
        def _pref():
            gat(i0 + jnp.int32(2), rows0, sem0)

        gwait(i1, rows1, sem1)
        pltpu.sync_copy(rows1, acc.at[didx_h.at[i1]], add=True)
        return carry

    for half in range(2):
        pltpu.sync_copy(src_hbm.at[w, pl.ds(jnp.int32(half * nh), nh)], sidx_h)
        pltpu.sync_copy(dst_hbm.at[w, pl.ds(jnp.int32(half * nh), nh)], didx_h)
        gat(jnp.int32(0), rows0, sem0)
        lax.fori_loop(jnp.int32(0), jnp.int32(n2), body, jnp.int32(0))
    plsc.subcore_barrier()
    pltpu.sync_copy(acc.at[pl.ds(s * jnp.int32(rows_per_sub), rows_per_sub)],
                    out_hbm.at[c, pl.ds(s * jnp.int32(rows_per_sub), rows_per_sub)])


def _dinv_from_parts(degp_ref):
    deg = jnp.sum(degp_ref[...], axis=0) + 1.0
    return lax.rsqrt(deg)


def _mm_scale_kernel(x_ref, w_ref, degp_ref, o_ref):
    dinv = _dinv_from_parts(degp_ref)
    o_ref[...] = jnp.dot(x_ref[...], w_ref[...], preferred_element_type=F32, precision=lax.Precision.HIGHEST) * dinv


def _combine_kernel(aggp_ref, y_ref, degp_ref, b_ref, w_ref, o_ref):
    dinv = _dinv_from_parts(degp_ref)
    t = (aggp_ref[0] + aggp_ref[1] + y_ref[...]) * dinv + b_ref[...]
    h = jnp.maximum(t, 0.0)
    o_ref[...] = jnp.dot(h, w_ref[...], preferred_element_type=F32, precision=lax.Precision.HIGHEST) * dinv


def _final_kernel(aggp_ref, y_ref, degp_ref, b_ref, batch_ref, wout_ref,
                  bout_ref, o_ref, sums, cnts):
    i = pl.program_id(0)

    @pl.when(i == 0)
    def _init():
        sums[...] = jnp.zeros_like(sums)
        cnts[...] = jnp.zeros_like(cnts)

    dinv = _dinv_from_parts(degp_ref)
    t = (aggp_ref[0] + aggp_ref[1] + y_ref[...]) * dinv + b_ref[...]
    h = jnp.maximum(t, 0.0)
    oh = (batch_ref[...] == lax.broadcasted_iota(I32, (1, G), 1)).astype(F32)
    dn = (((0,), (0,)), ((), ()))
    sums[...] += lax.dot_general(oh, h, dn, preferred_element_type=F32, precision=lax.Precision.HIGHEST)
    cnts[...] += lax.dot_general(oh, jnp.ones_like(h), dn, preferred_element_type=F32, precision=lax.Precision.HIGHEST)

    @pl.when(i == pl.num_programs(0) - 1)
    def _fin():
        mean = sums[...] / jnp.maximum(cnts[...], 1.0)
        o_ref[...] = jnp.dot(mean, wout_ref[...], preferred_element_type=F32, precision=lax.Precision.HIGHEST) + bout_ref[...]


def _sc_mesh():
    return plsc.VectorSubcoreMesh(core_axis_name="c", subcore_axis_name="s",
                                  num_cores=NC, num_subcores=NS)


def _deg_parts(dst3d, npad):
    nch, cw = dst3d.shape[1], dst3d.shape[2]
    call = pl.kernel(
        functools.partial(_deg_body, nchunks=nch, cw=cw),
        out_type=jax.ShapeDtypeStruct((NC, NS, npad), F32),
        mesh=_sc_mesh(),
        scratch_types=[
            pltpu.VMEM((nch, cw), I32),
            pltpu.VMEM((npad,), F32),
        ],
        compiler_params=pltpu.CompilerParams(needs_layout_passes=False),
    )
    return call(dst3d)


def _agg_parts(src3a, dst3a, y, npad):
    h = y.shape[1]
    nch, ka = src3a.shape[1], src3a.shape[2]
    call = pl.kernel(
        functools.partial(_agg_body, nchunks=nch),
        out_type=jax.ShapeDtypeStruct((NC, npad, h), F32),
        mesh=_sc_mesh(),
        scratch_types=[
            pltpu.VMEM((nch // 2, ka), I32),
            pltpu.VMEM((nch // 2, ka), I32),
            pltpu.VMEM((ka, h), F32),
            pltpu.VMEM((ka, h), F32),
            pltpu.VMEM_SHARED((npad, h), F32),
            pltpu.SemaphoreType.DMA,
            pltpu.SemaphoreType.DMA,
        ],
        compiler_params=pltpu.CompilerParams(needs_layout_passes=False),
    )
    return call(src3a, dst3a, y)


_Z = np.int32(0)


def _im_i0(i):
    return i, _Z


def _im_0i0(i):
    return _Z, i, _Z


def _im_00(i):
    return _Z, _Z


def kernel(x, edge_index, batch, W1, b1, W2, b2, Wout, bout):
    N, D = x.shape
    H = W1.shape[1]
    E = edge_index.shape[1]
    NPAD = (N // 1024 + 1) * 1024
    R = 1024
    grid = NPAD // R
    ew = E // NW
    nch = 80
    ka = ew // nch
    src3a = edge_index[0].astype(I32).reshape(NW, nch, ka)
    dst3a = edge_index[1].astype(I32).reshape(NW, nch, ka)
    dst3d = edge_index[1].astype(I32).reshape(NW, ka, nch)

    x32 = x.astype(F32)
    x_p = jnp.concatenate([x32, jnp.zeros((NPAD - N, D), F32)])
    batch_p = jnp.concatenate([batch.astype(I32),
                               jnp.full((NPAD - N,), G, I32)]).reshape(NPAD, 1)
    W1f = W1.astype(F32)
    W2f = W2.astype(F32)
    Woutf = Wout.astype(F32)
    b1r = b1.astype(F32).reshape(1, H)
    b2r = b2.astype(F32).reshape(1, H)
    boutr = bout.astype(F32).reshape(1, 1)

    degparts = _deg_parts(dst3d, NPAD).reshape(NW, NPAD, 1)

    y1 = pl.pallas_call(
        _mm_scale_kernel,
        grid=(grid,),
        in_specs=[
            pl.BlockSpec((R, D), _im_i0),
            pl.BlockSpec((D, H), _im_00),
            pl.BlockSpec((NW, R, 1), _im_0i0),
        ],
        out_specs=pl.BlockSpec((R, H), _im_i0),
        out_shape=jax.ShapeDtypeStruct((NPAD, H), F32),
    )(x_p, W1f, degparts)

    agg1 = _agg_parts(src3a, dst3a, y1, NPAD)

    y2 = pl.pallas_call(
        _combine_kernel,
        grid=(grid,),
        in_specs=[
            pl.BlockSpec((NC, R, H), _im_0i0),
            pl.BlockSpec((R, H), _im_i0),
            pl.BlockSpec((NW, R, 1), _im_0i0),
            pl.BlockSpec((1, H), _im_00),
            pl.BlockSpec((H, H), _im_00),
        ],
        out_specs=pl.BlockSpec((R, H), _im_i0),
        out_shape=jax.ShapeDtypeStruct((NPAD, H), F32),
    )(agg1, y1, degparts, b1r, W2f)

    agg2 = _agg_parts(src3a, dst3a, y2, NPAD)

    out = pl.pallas_call(
        _final_kernel,
        grid=(grid,),
        in_specs=[
            pl.BlockSpec((NC, R, H), _im_0i0),
            pl.BlockSpec((R, H), _im_i0),
            pl.BlockSpec((NW, R, 1), _im_0i0),
            pl.BlockSpec((1, H), _im_00),
            pl.BlockSpec((R, 1), _im_i0),
            pl.BlockSpec((H, 1), _im_00),
            pl.BlockSpec((1, 1), _im_00),
        ],
        out_specs=pl.BlockSpec((G, 1), _im_00),
        out_shape=jax.ShapeDtypeStruct((G, 1), F32),
        scratch_shapes=[
            pltpu.VMEM((G, H), F32),
            pltpu.VMEM((G, H), F32),
        ],
    )(agg2, y2, degparts, b2r, batch_p, Woutf, boutr)

    return out.astype(jnp.float64)


# speedup vs baseline: 4.2236x; 1.6910x over previous
import functools

import jax
import jax.numpy as jnp
import numpy as np
from jax import lax
from jax.experimental import pallas as pl
from jax.experimental.pallas import tpu as pltpu
from jax.experimental.pallas import tpu_sc as plsc

F32 = jnp.float32
I32 = jnp.int32

NC = 2
NS = 16
NW = NC * NS
K = 128
G = 64


def _deg_body(dst_hbm, out_hbm, idx_v, hist, *, nchunks, cw):
    c = lax.axis_index("c")
    s = lax.axis_index("s")
    w = c * jnp.int32(NS) + s
    npad = hist.shape[0]
    pltpu.sync_copy(dst_hbm.at[w], idx_v)

    def zero(r, carry):
        hist[pl.ds(r * jnp.int32(16), 16)] = jnp.zeros((16,), F32)
        return carry

    lax.fori_loop(jnp.int32(0), jnp.int32(npad // 16), zero, jnp.int32(0))

    def body(i, carry):
        for k in range(cw // 16):
            v = idx_v[i, pl.ds(jnp.int32(k * 16), 16)]
            plsc.addupdate_scatter(hist, [v], jnp.ones((16,), F32))
        return carry

    lax.fori_loop(jnp.int32(0), jnp.int32(nchunks), body, jnp.int32(0))
    pltpu.sync_copy(hist, out_hbm.at[c, s])


def _agg_body(src_hbm, dst_hbm, y_hbm, out_hbm,
              sidx_h, didx_h, rows0, rows1, acc, sem0, sem1, *, nchunks):
    c = lax.axis_index("c")
    s = lax.axis_index("s")
    w = c * jnp.int32(NS) + s
    rows_per_sub = acc.shape[0] // NS
    ka = rows0.shape[0]

    def zrow(r, carry):
        for j in range(rows0.shape[1] // 16):
            rows0[r, pl.ds(jnp.int32(j * 16), 16)] = jnp.zeros((16,), F32)
        return carry

    lax.fori_loop(jnp.int32(0), jnp.int32(ka), zrow, jnp.int32(0))
    nfull = rows_per_sub // ka
    for b in range(nfull):
        pltpu.sync_copy(
            rows0, acc.at[pl.ds(s * jnp.int32(rows_per_sub) + jnp.int32(b * ka), ka)])
    tail = rows_per_sub - nfull * ka
    if tail:
        pltpu.sync_copy(
            rows0.at[pl.ds(0, tail)],
            acc.at[pl.ds(s * jnp.int32(rows_per_sub) + jnp.int32(nfull * ka), tail)])
    plsc.subcore_barrier()
    nh = nchunks // 2
    n2 = nh // 2

    def gat(i, rows, sem):
        pltpu.async_copy(y_hbm.at[sidx_h.at[i]], rows, sem)

    def gwait(i, rows, sem):
        pltpu.make_async_copy(y_hbm.at[sidx_h.at[i]], rows, sem).wait()

    def body(i2, carry):
        i0 = i2 * jnp.int32(2)
        i1 = i0 + jnp.int32(1)
        gat(i1, rows1, sem1)
        gwait(i0, rows0, sem0)
        pltpu.sync_copy(rows0, acc.at[didx_h.at[i0]], add=True)

        @pl.when(i2 < jnp.int32(n2 - 1))
        def _pref():
            gat(i0 + jnp.int32(2), rows0, sem0)

        gwait(i1, rows1, sem1)
        pltpu.sync_copy(rows1, acc.at[didx_h.at[i1]], add=True)
        return carry

    for half in range(2):
        pltpu.sync_copy(src_hbm.at[w, pl.ds(jnp.int32(half * nh), nh)], sidx_h)
        pltpu.sync_copy(dst_hbm.at[w, pl.ds(jnp.int32(half * nh), nh)], didx_h)
        gat(jnp.int32(0), rows0, sem0)
        lax.fori_loop(jnp.int32(0), jnp.int32(n2), body, jnp.int32(0))
    plsc.subcore_barrier()
    pltpu.sync_copy(acc.at[pl.ds(s * jnp.int32(rows_per_sub), rows_per_sub)],
                    out_hbm.at[c, pl.ds(s * jnp.int32(rows_per_sub), rows_per_sub)])


def _dinv_from_parts(degp_ref):
    deg = jnp.sum(degp_ref[...], axis=0) + 1.0
    return lax.rsqrt(deg)[:, None]


def _mm_scale_kernel(x_ref, w_ref, degp_ref, o_ref):
    dinv = _dinv_from_parts(degp_ref)
    o_ref[...] = jnp.dot(x_ref[...], w_ref[...], preferred_element_type=F32, precision=lax.Precision.HIGHEST) * dinv


def _combine_kernel(aggp_ref, y_ref, degp_ref, b_ref, w_ref, o_ref):
    dinv = _dinv_from_parts(degp_ref)
    t = (aggp_ref[0] + aggp_ref[1] + y_ref[...]) * dinv + b_ref[...]
    h = jnp.maximum(t, 0.0)
    o_ref[...] = jnp.dot(h, w_ref[...], preferred_element_type=F32, precision=lax.Precision.HIGHEST) * dinv


def _final_kernel(aggp_ref, y_ref, degp_ref, b_ref, batch_ref, wout_ref,
                  bout_ref, o_ref, sums, cnts):
    i = pl.program_id(0)

    @pl.when(i == 0)
    def _init():
        sums[...] = jnp.zeros_like(sums)
        cnts[...] = jnp.zeros_like(cnts)

    dinv = _dinv_from_parts(degp_ref)
    t = (aggp_ref[0] + aggp_ref[1] + y_ref[...]) * dinv + b_ref[...]
    h = jnp.maximum(t, 0.0)
    oh = (batch_ref[...] == lax.broadcasted_iota(I32, (1, G), 1)).astype(F32)
    dn = (((0,), (0,)), ((), ()))
    sums[...] += lax.dot_general(oh, h, dn, preferred_element_type=F32, precision=lax.Precision.HIGHEST)
    cnts[...] += lax.dot_general(oh, jnp.ones_like(h), dn, preferred_element_type=F32, precision=lax.Precision.HIGHEST)

    @pl.when(i == pl.num_programs(0) - 1)
    def _fin():
        mean = sums[...] / jnp.maximum(cnts[...], 1.0)
        o_ref[...] = jnp.dot(mean, wout_ref[...], preferred_element_type=F32, precision=lax.Precision.HIGHEST) + bout_ref[...]


def _sc_mesh():
    return plsc.VectorSubcoreMesh(core_axis_name="c", subcore_axis_name="s",
                                  num_cores=NC, num_subcores=NS)


def _deg_parts(dst3d, npad):
    nch, cw = dst3d.shape[1], dst3d.shape[2]
    call = pl.kernel(
        functools.partial(_deg_body, nchunks=nch, cw=cw),
        out_type=jax.ShapeDtypeStruct((NC, NS, npad), F32),
        mesh=_sc_mesh(),
        scratch_types=[
            pltpu.VMEM((nch, cw), I32),
            pltpu.VMEM((npad,), F32),
        ],
        compiler_params=pltpu.CompilerParams(needs_layout_passes=False),
    )
    return call(dst3d)


def _agg_parts(src3a, dst3a, y, npad):
    h = y.shape[1]
    nch, ka = src3a.shape[1], src3a.shape[2]
    call = pl.kernel(
        functools.partial(_agg_body, nchunks=nch),
        out_type=jax.ShapeDtypeStruct((NC, npad, h), F32),
        mesh=_sc_mesh(),
        scratch_types=[
            pltpu.VMEM((nch // 2, ka), I32),
            pltpu.VMEM((nch // 2, ka), I32),
            pltpu.VMEM((ka, h), F32),
            pltpu.VMEM((ka, h), F32),
            pltpu.VMEM_SHARED((npad, h), F32),
            pltpu.SemaphoreType.DMA,
            pltpu.SemaphoreType.DMA,
        ],
        compiler_params=pltpu.CompilerParams(needs_layout_passes=False),
    )
    return call(src3a, dst3a, y)


_Z = np.int32(0)


def _im_i0(i):
    return i, _Z


def _im_0i0(i):
    return _Z, i, _Z


def _im_00(i):
    return _Z, _Z


def _im_i0x(i):
    return _Z, i


def kernel(x, edge_index, batch, W1, b1, W2, b2, Wout, bout):
    N, D = x.shape
    H = W1.shape[1]
    E = edge_index.shape[1]
    NPAD = (N // 1024 + 1) * 1024
    R = 1024
    grid = NPAD // R
    ew = E // NW
    nch = 80
    ka = ew // nch
    src3a = edge_index[0].astype(I32).reshape(NW, nch, ka)
    dst3a = edge_index[1].astype(I32).reshape(NW, nch, ka)
    dst3d = edge_index[1].astype(I32).reshape(NW, ka, nch)

    x32 = x.astype(F32)
    x_p = jnp.concatenate([x32, jnp.zeros((NPAD - N, D), F32)])
    batch_p = jnp.concatenate([batch.astype(I32),
                               jnp.full((NPAD - N,), G, I32)]).reshape(NPAD, 1)
    W1f = W1.astype(F32)
    W2f = W2.astype(F32)
    Woutf = Wout.astype(F32)
    b1r = b1.astype(F32).reshape(1, H)
    b2r = b2.astype(F32).reshape(1, H)
    boutr = bout.astype(F32).reshape(1, 1)

    degparts = _deg_parts(dst3d, NPAD).reshape(NW, NPAD)

    y1 = pl.pallas_call(
        _mm_scale_kernel,
        grid=(grid,),
        in_specs=[
            pl.BlockSpec((R, D), _im_i0),
            pl.BlockSpec((D, H), _im_00),
            pl.BlockSpec((NW, R), _im_i0x),
        ],
        out_specs=pl.BlockSpec((R, H), _im_i0),
        out_shape=jax.ShapeDtypeStruct((NPAD, H), F32),
    )(x_p, W1f, degparts)

    agg1 = _agg_parts(src3a, dst3a, y1, NPAD)

    y2 = pl.pallas_call(
        _combine_kernel,
        grid=(grid,),
        in_specs=[
            pl.BlockSpec((NC, R, H), _im_0i0),
            pl.BlockSpec((R, H), _im_i0),
            pl.BlockSpec((NW, R), _im_i0x),
            pl.BlockSpec((1, H), _im_00),
            pl.BlockSpec((H, H), _im_00),
        ],
        out_specs=pl.BlockSpec((R, H), _im_i0),
        out_shape=jax.ShapeDtypeStruct((NPAD, H), F32),
    )(agg1, y1, degparts, b1r, W2f)

    agg2 = _agg_parts(src3a, dst3a, y2, NPAD)

    out = pl.pallas_call(
        _final_kernel,
        grid=(grid,),
        in_specs=[
            pl.BlockSpec((NC, R, H), _im_0i0),
            pl.BlockSpec((R, H), _im_i0),
            pl.BlockSpec((NW, R), _im_i0x),
            pl.BlockSpec((1, H), _im_00),
            pl.BlockSpec((R, 1), _im_i0),
            pl.BlockSpec((H, 1), _im_00),
            pl.BlockSpec((1, 1), _im_00),
        ],
        out_specs=pl.BlockSpec((G, 1), _im_00),
        out_shape=jax.ShapeDtypeStruct((G, 1), F32),
        scratch_shapes=[
            pltpu.VMEM((G, H), F32),
            pltpu.VMEM((G, H), F32),
        ],
    )(agg2, y2, degparts, b2r, batch_p, Woutf, boutr)

    return out.astype(jnp.float64)
